# baseline (speedup 1.0000x reference)
import functools

import jax
import jax.numpy as jnp
import numpy as np
from jax import lax
from jax.experimental import pallas as pl
from jax.experimental.pallas import tpu as pltpu
from jax.experimental.pallas import tpu_sc as plsc

N = 10000
E = 320000
IN_CH = 128
HID = 64
NP = 10240
NW = 32
B = 128
CH = 79
EPAD = NW * CH * B
ROWS_PER_SUB = NP // 16
SR = 8
SC_COLS = NP // SR

_NEG = np.float32(-3.0e38)
_HI = lax.Precision.HIGHEST

_TL = (((0,), (0,)), ((), ()))
_TR = (((1,), (1,)), ((), ()))


def _tc_main_body(xp_ref, pT_ref, cntT_ref, H_ref,
                  Wz_ref, Uz_ref, Wr_ref, Ur_ref, Wh_ref, Uh_ref,
                  bz_ref, br_ref, bh_ref, Wl_ref,
                  xws_ref, dinv_ref, xwl_ref):
    f32 = jnp.float32
    pT = pT_ref[...]
    pn = jnp.sqrt(jnp.sum(pT * pT)) + f32(1e-12)
    xp = xp_ref[...]
    rows = [lax.dot_general(pT, xp[r * SC_COLS:(r + 1) * SC_COLS, :], _TR,
                            preferred_element_type=f32, precision=_HI)
            for r in range(SR)]
    sc0 = jnp.concatenate(rows, axis=0) * (f32(1.0) / pn)
    fi = (lax.broadcasted_iota(jnp.int32, (SR, SC_COLS), 0) * SC_COLS
          + lax.broadcasted_iota(jnp.int32, (SR, SC_COLS), 1))
    sc0 = jnp.where(fi < N, sc0, _NEG)

    lane_iota = lax.broadcasted_iota(jnp.int32, (1, IN_CH), 1)

    def topk_step(_, carry):
        sc, idxv, tvv, jj = carry
        m = jnp.max(sc)
        am = jnp.min(jnp.where(sc == m, fi, jnp.int32(2147483647)))
        tv = jnp.tanh(m)
        idxv = jnp.where(lane_iota == jj, am, idxv)
        tvv = jnp.where(lane_iota == jj, tv, tvv)
        sc = jnp.where(fi == am, _NEG, sc)
        return sc, idxv, tvv, jj + jnp.int32(1)

    idx0 = jnp.full((1, IN_CH), -1, dtype=jnp.int32)
    tv0 = jnp.zeros((1, IN_CH), dtype=f32)
    _, idxv, tvv, _ = lax.fori_loop(
        0, HID, topk_step, (sc0, idx0, tv0, jnp.int32(0)))

    row_iota = lax.broadcasted_iota(jnp.int32, (NP, IN_CH), 0)
    St = jnp.where(row_iota == idxv, tvv, f32(0.0))
    xt = lax.dot_general(xp, St, _TL, preferred_element_type=f32,
                         precision=_HI)

    H = H_ref[...]
    bz = bz_ref[...]
    br = br_ref[...]
    bh = bh_ref[...]
    Z = jax.nn.sigmoid(lax.dot(Wz_ref[...], xt, preferred_element_type=f32, precision=_HI)
                       + lax.dot(Uz_ref[...], H, preferred_element_type=f32, precision=_HI) + bz)
    R = jax.nn.sigmoid(lax.dot(Wr_ref[...], xt, preferred_element_type=f32, precision=_HI)
                       + lax.dot(Ur_ref[...], H, preferred_element_type=f32, precision=_HI) + br)
    Ht = jnp.tanh(lax.dot(Wh_ref[...], xt, preferred_element_type=f32, precision=_HI)
                  + lax.dot(Uh_ref[...], R * H, preferred_element_type=f32, precision=_HI) + bh)
    Wt = ((f32(1.0) - Z) * H + Z * Ht)[:, :HID]

    xw = lax.dot(xp, Wt, preferred_element_type=f32, precision=_HI)
    cnt = cntT_ref[...]
    deg = cnt[:, 0:1] + cnt[:, 1:2] + f32(1.0)
    dinv = lax.rsqrt(deg)
    xws = xw * dinv
    xws_ref[...] = xws
    dinv_ref[...] = dinv
    xwl_ref[...] = lax.dot(xws, Wl_ref[...], preferred_element_type=f32,
                           precision=_HI) * dinv


def _tc_main(xp, pT, cntT, H_pad, Wz, Uz, Wr, Ur, Wh, Uh, bz, br, bh, Wl):
    return pl.pallas_call(
        _tc_main_body,
        out_shape=[
            jax.ShapeDtypeStruct((NP, HID), jnp.float32),
            jax.ShapeDtypeStruct((NP, 1), jnp.float32),
            jax.ShapeDtypeStruct((NP, 1), jnp.float32),
        ],
    )(xp, pT, cntT, H_pad, Wz, Uz, Wr, Ur, Wh, Uh, bz, br, bh, Wl)


def _tc_out_body(acc_ref, xwl_ref, dinv_ref, Wl_ref, bl_ref, out_ref):
    f32 = jnp.float32
    s = acc_ref[0] + acc_ref[1]
    out_ref[...] = (lax.dot(s, Wl_ref[...], preferred_element_type=f32,
                            precision=_HI)
                    * dinv_ref[...] + xwl_ref[...] + bl_ref[...])


def _tc_out(acc, xwl, dinv, Wl, bl2):
    return pl.pallas_call(
        _tc_out_body,
        out_shape=jax.ShapeDtypeStruct((NP, 1), jnp.float32),
    )(acc, xwl, dinv, Wl, bl2)


def _sc_mesh():
    return plsc.VectorSubcoreMesh(core_axis_name="c", subcore_axis_name="s")


def _sc_deg_body(dst_hbm, out_hbm, idx_v, ones_v, zer_v, deg_sh, sem):
    c = lax.axis_index("c")
    s = lax.axis_index("s")
    wid = s * 2 + c
    for i in range(B // 16):
        ones_v[pl.ds(i * 16, 16)] = jnp.ones((16,), jnp.float32)
    for i in range(ROWS_PER_SUB // 16):
        zer_v[pl.ds(i * 16, 16)] = jnp.zeros((16,), jnp.float32)
    pltpu.sync_copy(zer_v, deg_sh.at[pl.ds(s * ROWS_PER_SUB, ROWS_PER_SUB)])
    plsc.subcore_barrier()
    pltpu.sync_copy(dst_hbm.at[wid], idx_v)

    def chunk(j, carry):
        pltpu.sync_copy(ones_v, deg_sh.at[idx_v.at[j]], add=True)
        return carry

    lax.fori_loop(0, CH, chunk, 0)
    plsc.subcore_barrier()
    pltpu.sync_copy(deg_sh.at[pl.ds(s * ROWS_PER_SUB, ROWS_PER_SUB)],
                    out_hbm.at[c, pl.ds(s * ROWS_PER_SUB, ROWS_PER_SUB)])


def _sc_deg(dst_rs):
    kfn = functools.partial(
        pl.kernel,
        mesh=_sc_mesh(),
        out_type=jax.ShapeDtypeStruct((2, NP), jnp.float32),
        scratch_types=[
            pltpu.VMEM((CH, B), jnp.int32),
            pltpu.VMEM((B,), jnp.float32),
            pltpu.VMEM((ROWS_PER_SUB,), jnp.float32),
            pltpu.VMEM_SHARED((NP,), jnp.float32),
            pltpu.SemaphoreType.DMA,
        ],
    )(_sc_deg_body)
    return kfn(dst_rs)


ZROWS = 40


def _sc_edge_body(src_hbm, dst_hbm, xws_hbm, out_hbm,
                  src_v, dst_v, rows0, rows1, rows2, rows3, rows4, rows5,
                  zer_v, acc_sh,
                  g0, g1, g2, g3, g4, g5, s0, s1, s2, s3, s4, s5):
    c = lax.axis_index("c")
    s = lax.axis_index("s")
    wid = s * 2 + c
    for i in range(ZROWS):
        for k in range(HID // 16):
            zer_v[i, pl.ds(k * 16, 16)] = jnp.zeros((16,), jnp.float32)
    for t in range(ROWS_PER_SUB // ZROWS):
        pltpu.sync_copy(zer_v,
                        acc_sh.at[pl.ds(s * ROWS_PER_SUB + t * ZROWS, ZROWS)])
    plsc.subcore_barrier()
    pltpu.sync_copy(src_hbm.at[wid], src_v)
    pltpu.sync_copy(dst_hbm.at[wid], dst_v)

    rows = (rows0, rows1, rows2, rows3, rows4, rows5)
    gsem = (g0, g1, g2, g3, g4, g5)
    ssem = (s0, s1, s2, s3, s4, s5)
    NB = 6

    def g_start(b, j):
        pltpu.async_copy(xws_hbm.at[src_v.at[j]], rows[b], gsem[b])

    def g_wait(b, j):
        pltpu.make_async_copy(xws_hbm.at[src_v.at[j]], rows[b], gsem[b]).wait()

    def s_start(b, j):
        pltpu.async_copy(rows[b], acc_sh.at[dst_v.at[j]], ssem[b], add=True)

    def s_wait(b, j):
        pltpu.make_async_copy(rows[b], acc_sh.at[dst_v.at[j]],
                              ssem[b]).wait()

    LEAD = NB // 2
    for j in range(LEAD):
        g_start(j, jnp.int32(j))

    def step(_, j0):
        for b in range(NB):
            j = j0 + jnp.int32(b)
            jn = j + jnp.int32(LEAD)
            sn = (b + LEAD) % NB
            g_wait(b, j)

            @pl.when((jn < CH) & (jn >= NB))
            def _():
                s_wait(sn, j)

            @pl.when(jn < CH)
            def _():
                g_start(sn, jn)
            s_start(b, j)
        return j0 + jnp.int32(NB)

    lax.fori_loop(0, CH // NB, step, jnp.int32(0))
    for j in range(CH - CH % NB, CH):
        b = j % NB
        jn = j + LEAD
        if jn < CH:
            if jn >= NB:
                s_wait(jn % NB, jnp.int32(j))
            g_start(jn % NB, jnp.int32(jn))
        g_wait(b, jnp.int32(j))
        s_start(b, jnp.int32(j))
    drained = set(j + LEAD - NB for j in range(CH)
                  if (j + LEAD) < CH and (j + LEAD) >= NB)
    for j in range(CH):
        if j not in drained:
            s_wait(j % NB, jnp.int32(j))

    plsc.subcore_barrier()
    pltpu.sync_copy(acc_sh.at[pl.ds(s * ROWS_PER_SUB, ROWS_PER_SUB)],
                    out_hbm.at[c, pl.ds(s * ROWS_PER_SUB, ROWS_PER_SUB)])


def _sc_edge(src_rs, dst_rs, xws):
    kfn = functools.partial(
        pl.kernel,
        mesh=_sc_mesh(),
        compiler_params=pltpu.CompilerParams(use_tc_tiling_on_sc=False),
        out_type=jax.ShapeDtypeStruct((2, NP, HID), jnp.float32),
        scratch_types=[
            pltpu.VMEM((CH, B), jnp.int32),
            pltpu.VMEM((CH, B), jnp.int32),
            pltpu.VMEM((B, HID), jnp.float32),
            pltpu.VMEM((B, HID), jnp.float32),
            pltpu.VMEM((B, HID), jnp.float32),
            pltpu.VMEM((B, HID), jnp.float32),
            pltpu.VMEM((B, HID), jnp.float32),
            pltpu.VMEM((B, HID), jnp.float32),
            pltpu.VMEM((ZROWS, HID), jnp.float32),
            pltpu.VMEM_SHARED((NP, HID), jnp.float32),
        ] + [pltpu.SemaphoreType.DMA] * 12,
    )(_sc_edge_body)
    return kfn(src_rs, dst_rs, xws)


def kernel(x, edge_index, p, W_init, Wz, Uz, Wr, Ur, Wh, Uh, bz, br, bh, Wl, bl):
    f32 = jnp.float32
    xf = x.astype(f32)
    xp = jnp.zeros((NP, IN_CH), dtype=f32).at[:N, :].set(xf)
    pT = p.astype(f32).reshape(1, IN_CH)

    ei = edge_index.astype(jnp.int32)
    pad_idx = (jnp.arange(EPAD - E, dtype=jnp.int32) % (NP - N)) + N
    src_rs = jnp.concatenate([ei[0], pad_idx]).reshape(NW, CH, B)
    dst_rs = jnp.concatenate([ei[1], pad_idx]).reshape(NW, CH, B)

    H_pad = jnp.zeros((IN_CH, IN_CH), dtype=f32).at[:, :HID].set(
        W_init.astype(f32))

    counts = _sc_deg(dst_rs)
    xws, dinv, xwl = _tc_main(xp, pT, counts.T, H_pad,
                              Wz.astype(f32), Uz.astype(f32), Wr.astype(f32),
                              Ur.astype(f32), Wh.astype(f32), Uh.astype(f32),
                              bz.astype(f32), br.astype(f32), bh.astype(f32),
                              Wl.astype(f32))

    acc = _sc_edge(src_rs, dst_rs, xws)

    out = _tc_out(acc, xwl, dinv, Wl.astype(f32), bl.astype(f32).reshape(1, 1))
    out_dtype = jnp.promote_types(x.dtype, Wl.dtype)
    return out[:N, 0].astype(out_dtype)


# speedup vs baseline: 1.0648x; 1.0648x over previous
import functools

import jax
import jax.numpy as jnp
import numpy as np
from jax import lax
from jax.experimental import pallas as pl
from jax.experimental.pallas import tpu as pltpu
from jax.experimental.pallas import tpu_sc as plsc

N = 10000
E = 320000
IN_CH = 128
HID = 64
NP = 10240
NW = 32
B = 128
CH = 79
EPAD = NW * CH * B
ROWS_PER_SUB = NP // 16
SR = 8
SC_COLS = NP // SR

_NEG = np.float32(-3.0e38)
_HI = lax.Precision.HIGHEST

_TL = (((0,), (0,)), ((), ()))
_TR = (((1,), (1,)), ((), ()))


def _tc_main_body(xp_ref, pT_ref, cntT_ref, H_ref,
                  Wz_ref, Uz_ref, Wr_ref, Ur_ref, Wh_ref, Uh_ref,
                  bz_ref, br_ref, bh_ref,
                  xws_ref, dinv_ref):
    f32 = jnp.float32
    pT = pT_ref[...]
    pn = jnp.sqrt(jnp.sum(pT * pT)) + f32(1e-12)
    xp = xp_ref[...]
    rows = [lax.dot_general(pT, xp[r * SC_COLS:(r + 1) * SC_COLS, :], _TR,
                            preferred_element_type=f32, precision=_HI)
            for r in range(SR)]
    sc0 = jnp.concatenate(rows, axis=0) * (f32(1.0) / pn)
    fi = (lax.broadcasted_iota(jnp.int32, (SR, SC_COLS), 0) * SC_COLS
          + lax.broadcasted_iota(jnp.int32, (SR, SC_COLS), 1))
    sc0 = jnp.where(fi < N, sc0, _NEG)

    lane_iota = lax.broadcasted_iota(jnp.int32, (1, IN_CH), 1)

    def topk_step(_, carry):
        sc, idxv, tvv, jj = carry
        m = jnp.max(sc)
        am = jnp.min(jnp.where(sc == m, fi, jnp.int32(2147483647)))
        tv = jnp.tanh(m)
        idxv = jnp.where(lane_iota == jj, am, idxv)
        tvv = jnp.where(lane_iota == jj, tv, tvv)
        sc = jnp.where(fi == am, _NEG, sc)
        return sc, idxv, tvv, jj + jnp.int32(1)

    idx0 = jnp.full((1, IN_CH), -1, dtype=jnp.int32)
    tv0 = jnp.zeros((1, IN_CH), dtype=f32)
    _, idxv, tvv, _ = lax.fori_loop(
        0, HID, topk_step, (sc0, idx0, tv0, jnp.int32(0)))

    row_iota = lax.broadcasted_iota(jnp.int32, (NP, IN_CH), 0)
    St = jnp.where(row_iota == idxv, tvv, f32(0.0))
    xt = lax.dot_general(xp, St, _TL, preferred_element_type=f32,
                         precision=_HI)

    H = H_ref[...]
    bz = bz_ref[...]
    br = br_ref[...]
    bh = bh_ref[...]
    Z = jax.nn.sigmoid(lax.dot(Wz_ref[...], xt, preferred_element_type=f32, precision=_HI)
                       + lax.dot(Uz_ref[...], H, preferred_element_type=f32, precision=_HI) + bz)
    R = jax.nn.sigmoid(lax.dot(Wr_ref[...], xt, preferred_element_type=f32, precision=_HI)
                       + lax.dot(Ur_ref[...], H, preferred_element_type=f32, precision=_HI) + br)
    Ht = jnp.tanh(lax.dot(Wh_ref[...], xt, preferred_element_type=f32, precision=_HI)
                  + lax.dot(Uh_ref[...], R * H, preferred_element_type=f32, precision=_HI) + bh)
    Wt = ((f32(1.0) - Z) * H + Z * Ht)[:, :HID]

    xw = lax.dot(xp, Wt, preferred_element_type=f32, precision=_HI)
    cnt = cntT_ref[...]
    deg = cnt[:, 0:1] + cnt[:, 1:2] + f32(1.0)
    dinv = lax.rsqrt(deg)
    xws_ref[...] = xw * dinv
    dinv_ref[...] = dinv


def _tc_main(xp, pT, cntT, H_pad, Wz, Uz, Wr, Ur, Wh, Uh, bz, br, bh):
    return pl.pallas_call(
        _tc_main_body,
        out_shape=[
            jax.ShapeDtypeStruct((NP, HID), jnp.float32),
            jax.ShapeDtypeStruct((NP, 1), jnp.float32),
        ],
    )(xp, pT, cntT, H_pad, Wz, Uz, Wr, Ur, Wh, Uh, bz, br, bh)


def _tc_out_body(acc_ref, xws_ref, dinv_ref, Wl_ref, bl_ref, out_ref):
    f32 = jnp.float32
    s = acc_ref[0] + acc_ref[1] + xws_ref[...]
    out_ref[...] = (lax.dot(s, Wl_ref[...], preferred_element_type=f32,
                            precision=_HI)
                    * dinv_ref[...] + bl_ref[...])


def _tc_out(acc, xws, dinv, Wl, bl2):
    return pl.pallas_call(
        _tc_out_body,
        out_shape=jax.ShapeDtypeStruct((NP, 1), jnp.float32),
    )(acc, xws, dinv, Wl, bl2)


def _sc_mesh():
    return plsc.VectorSubcoreMesh(core_axis_name="c", subcore_axis_name="s")


def _sc_deg_body(dst_hbm, out_hbm, idx_v, ones_v, zer_v, deg_sh, sem):
    c = lax.axis_index("c")
    s = lax.axis_index("s")
    wid = s * 2 + c
    for i in range(B // 16):
        ones_v[pl.ds(i * 16, 16)] = jnp.ones((16,), jnp.float32)
    for i in range(ROWS_PER_SUB // 16):
        zer_v[pl.ds(i * 16, 16)] = jnp.zeros((16,), jnp.float32)
    pltpu.sync_copy(zer_v, deg_sh.at[pl.ds(s * ROWS_PER_SUB, ROWS_PER_SUB)])
    plsc.subcore_barrier()
    pltpu.sync_copy(dst_hbm.at[wid], idx_v)

    def chunk(j, carry):
        pltpu.sync_copy(ones_v, deg_sh.at[idx_v.at[j]], add=True)
        return carry

    lax.fori_loop(0, CH, chunk, 0)
    plsc.subcore_barrier()
    pltpu.sync_copy(deg_sh.at[pl.ds(s * ROWS_PER_SUB, ROWS_PER_SUB)],
                    out_hbm.at[c, pl.ds(s * ROWS_PER_SUB, ROWS_PER_SUB)])


def _sc_deg(dst_rs):
    kfn = functools.partial(
        pl.kernel,
        mesh=_sc_mesh(),
        out_type=jax.ShapeDtypeStruct((2, NP), jnp.float32),
        scratch_types=[
            pltpu.VMEM((CH, B), jnp.int32),
            pltpu.VMEM((B,), jnp.float32),
            pltpu.VMEM((ROWS_PER_SUB,), jnp.float32),
            pltpu.VMEM_SHARED((NP,), jnp.float32),
            pltpu.SemaphoreType.DMA,
        ],
    )(_sc_deg_body)
    return kfn(dst_rs)


ZROWS = 40


def _sc_edge_body(src_hbm, dst_hbm, xws_hbm, out_hbm,
                  src_v, dst_v, rows0, rows1, rows2, rows3, rows4, rows5,
                  zer_v, acc_sh,
                  g0, g1, g2, g3, g4, g5, s0, s1, s2, s3, s4, s5):
    c = lax.axis_index("c")
    s = lax.axis_index("s")
    wid = s * 2 + c
    for i in range(ZROWS):
        for k in range(HID // 16):
            zer_v[i, pl.ds(k * 16, 16)] = jnp.zeros((16,), jnp.float32)
    for t in range(ROWS_PER_SUB // ZROWS):
        pltpu.sync_copy(zer_v,
                        acc_sh.at[pl.ds(s * ROWS_PER_SUB + t * ZROWS, ZROWS)])
    plsc.subcore_barrier()
    pltpu.sync_copy(src_hbm.at[wid], src_v)
    pltpu.sync_copy(dst_hbm.at[wid], dst_v)

    rows = (rows0, rows1, rows2, rows3, rows4, rows5)
    gsem = (g0, g1, g2, g3, g4, g5)
    ssem = (s0, s1, s2, s3, s4, s5)
    NB = 6

    def g_start(b, j):
        pltpu.async_copy(xws_hbm.at[src_v.at[j]], rows[b], gsem[b])

    def g_wait(b, j):
        pltpu.make_async_copy(xws_hbm.at[src_v.at[j]], rows[b], gsem[b]).wait()

    def s_start(b, j):
        pltpu.async_copy(rows[b], acc_sh.at[dst_v.at[j]], ssem[b], add=True)

    def s_wait(b, j):
        pltpu.make_async_copy(rows[b], acc_sh.at[dst_v.at[j]],
                              ssem[b]).wait()

    LEAD = NB // 2
    for j in range(LEAD):
        g_start(j, jnp.int32(j))

    def step(_, j0):
        for b in range(NB):
            j = j0 + jnp.int32(b)
            jn = j + jnp.int32(LEAD)
            sn = (b + LEAD) % NB
            g_wait(b, j)

            @pl.when((jn < CH) & (jn >= NB))
            def _():
                s_wait(sn, j)

            @pl.when(jn < CH)
            def _():
                g_start(sn, jn)
            s_start(b, j)
        return j0 + jnp.int32(NB)

    lax.fori_loop(0, CH // NB, step, jnp.int32(0))
    for j in range(CH - CH % NB, CH):
        b = j % NB
        jn = j + LEAD
        if jn < CH:
            if jn >= NB:
                s_wait(jn % NB, jnp.int32(j))
            g_start(jn % NB, jnp.int32(jn))
        g_wait(b, jnp.int32(j))
        s_start(b, jnp.int32(j))
    drained = set(j + LEAD - NB for j in range(CH)
                  if (j + LEAD) < CH and (j + LEAD) >= NB)
    for j in range(CH):
        if j not in drained:
            s_wait(j % NB, jnp.int32(j))

    plsc.subcore_barrier()
    pltpu.sync_copy(acc_sh.at[pl.ds(s * ROWS_PER_SUB, ROWS_PER_SUB)],
                    out_hbm.at[c, pl.ds(s * ROWS_PER_SUB, ROWS_PER_SUB)])


def _sc_edge(src_rs, dst_rs, xws):
    kfn = functools.partial(
        pl.kernel,
        mesh=_sc_mesh(),
        compiler_params=pltpu.CompilerParams(use_tc_tiling_on_sc=False),
        out_type=jax.ShapeDtypeStruct((2, NP, HID), jnp.float32),
        scratch_types=[
            pltpu.VMEM((CH, B), jnp.int32),
            pltpu.VMEM((CH, B), jnp.int32),
            pltpu.VMEM((B, HID), jnp.float32),
            pltpu.VMEM((B, HID), jnp.float32),
            pltpu.VMEM((B, HID), jnp.float32),
            pltpu.VMEM((B, HID), jnp.float32),
            pltpu.VMEM((B, HID), jnp.float32),
            pltpu.VMEM((B, HID), jnp.float32),
            pltpu.VMEM((ZROWS, HID), jnp.float32),
            pltpu.VMEM_SHARED((NP, HID), jnp.float32),
        ] + [pltpu.SemaphoreType.DMA] * 12,
    )(_sc_edge_body)
    return kfn(src_rs, dst_rs, xws)


def kernel(x, edge_index, p, W_init, Wz, Uz, Wr, Ur, Wh, Uh, bz, br, bh, Wl, bl):
    f32 = jnp.float32
    xf = x.astype(f32)
    xp = jnp.zeros((NP, IN_CH), dtype=f32).at[:N, :].set(xf)
    pT = p.astype(f32).reshape(1, IN_CH)

    ei = edge_index.astype(jnp.int32)
    pad_idx = (jnp.arange(EPAD - E, dtype=jnp.int32) % (NP - N)) + N
    src_rs = jnp.concatenate([ei[0], pad_idx]).reshape(NW, CH, B)
    dst_rs = jnp.concatenate([ei[1], pad_idx]).reshape(NW, CH, B)

    H_pad = jnp.zeros((IN_CH, IN_CH), dtype=f32).at[:, :HID].set(
        W_init.astype(f32))

    counts = _sc_deg(dst_rs)
    xws, dinv = _tc_main(xp, pT, counts.T, H_pad,
                         Wz.astype(f32), Uz.astype(f32), Wr.astype(f32),
                         Ur.astype(f32), Wh.astype(f32), Uh.astype(f32),
                         bz.astype(f32), br.astype(f32), bh.astype(f32))

    acc = _sc_edge(src_rs, dst_rs, xws)

    out = _tc_out(acc, xws, dinv, Wl.astype(f32), bl.astype(f32).reshape(1, 1))
    out_dtype = jnp.promote_types(x.dtype, Wl.dtype)
    return out[:N, 0].astype(out_dtype)
